# baseline (speedup 1.0000x reference)
import functools

import jax
import jax.numpy as jnp
from jax import lax
from jax.experimental import pallas as pl
from jax.experimental.pallas import tpu as pltpu
from jax.experimental.pallas import tpu_sc as plsc

N_USERS = 60000
N_ITEMS = 40000
NT = N_USERS + N_ITEMS
D = 32
NNZ = 1600000
BATCH = 4096
N_LAYERS = 3

NC = 2
NS = 16
NW = NC * NS
RPT = NT // NW
ACC_ROWS = RPT + 11
DUMMY = RPT + 3

CHUNK = 384
SW = 128
NSTREAM = CHUNK // SW
EROWS = NNZ // SW

_mesh = plsc.VectorSubcoreMesh(core_axis_name="core", subcore_axis_name="subcore")
_cparams = pltpu.CompilerParams(needs_layout_passes=False,
                                use_tc_tiling_on_sc=False)

_f32 = jnp.float32
_i32 = jnp.int32


@functools.partial(
    pl.kernel,
    out_type=jax.ShapeDtypeStruct((NT, D), _f32),
    mesh=_mesh,
    scratch_types=[
        pltpu.VMEM((48,), _i32),
        pltpu.VMEM((NSTREAM, SW), _i32),
        pltpu.VMEM((NSTREAM, SW), _i32),
        pltpu.VMEM((NSTREAM, SW), _i32),
        pltpu.VMEM((NSTREAM, SW), _i32),
        pltpu.VMEM((NSTREAM, SW), _f32),
        pltpu.VMEM((NSTREAM, SW), _f32),
        pltpu.VMEM((2 * CHUNK, D), _f32),
        pltpu.VMEM((ACC_ROWS, D), _f32),
        pltpu.SemaphoreType.DMA,
        pltpu.SemaphoreType.DMA,
        pltpu.SemaphoreType.DMA,
        pltpu.SemaphoreType.DMA,
    ],
    compiler_params=_cparams,
)
def _spmm_layer(table, col2d, row2d, val2d, bounds, out,
                bsm, colv0, colv1, rowv0, rowv1, valv0, valv1, gath, acc,
                sg0, sg1, si0, si1):
    c = lax.axis_index("core")
    s = lax.axis_index("subcore")
    w = c * NS + s
    pltpu.sync_copy(bounds, bsm)
    zeros16 = jnp.zeros((16,), _f32)
    iota16 = lax.iota(_i32, 16)
    b0 = bsm[pl.ds(0, 16)]
    b1 = bsm[pl.ds(16, 16)]
    b2 = bsm[pl.ds(32, 16)]

    def pick(vec, k):
        return jnp.sum(jnp.where(iota16 == k, vec, 0))

    e_lo = pick(b0, w) + pick(b1, w - 16)
    e_hi = pick(b0, w + 1) + pick(b1, w - 15) + pick(b2, w - 31)
    base = w * RPT

    @pl.loop(0, ACC_ROWS)
    def _(i):
        acc[i, pl.ds(0, 16)] = zeros16
        acc[i, pl.ds(16, 16)] = zeros16

    a0 = (e_lo // CHUNK) * CHUNK
    n_chunks = jnp.maximum(0, (e_hi - a0 + CHUNK - 1) // CHUNK)
    n_pairs = (n_chunks + 1) // 2

    sets = ((colv0, rowv0, valv0, si0), (colv1, rowv1, valv1, si1))
    sgs = (sg0, sg1)

    def rof(a):
        return jnp.minimum(a, NNZ - CHUNK) // SW

    def issue_idx(a, bset):
        colv, rowv, valv, sem_i = bset
        r = rof(a)
        pltpu.async_copy(col2d.at[pl.ds(r, NSTREAM)], colv, sem_i)
        pltpu.async_copy(row2d.at[pl.ds(r, NSTREAM)], rowv, sem_i)
        pltpu.async_copy(val2d.at[pl.ds(r, NSTREAM)], valv, sem_i)

    def wait_idx(bset):
        colv, rowv, valv, sem_i = bset
        pltpu.make_async_copy(col2d.at[pl.ds(0, NSTREAM)], colv, sem_i).wait()
        pltpu.make_async_copy(row2d.at[pl.ds(0, NSTREAM)], rowv, sem_i).wait()
        pltpu.make_async_copy(val2d.at[pl.ds(0, NSTREAM)], valv, sem_i).wait()

    def issue_gathers(bset, bank):
        colv = bset[0]
        for j in range(NSTREAM):
            pltpu.async_copy(table.at[colv.at[j]],
                             gath.at[pl.ds((bank * NSTREAM + j) * SW, SW)],
                             sgs[bank])

    def wait_gathers(bset, bank):
        colv = bset[0]
        for j in range(NSTREAM):
            pltpu.make_async_copy(table.at[colv.at[j]],
                                  gath.at[pl.ds((bank * NSTREAM + j) * SW, SW)],
                                  sgs[bank]).wait()

    def accum(a_cc, bset, bank):
        colv, rowv, valv, _ = bset
        a_eff = jnp.minimum(a_cc, NNZ - CHUNK)
        lo = jnp.maximum(e_lo, a_cc)
        boundary = (a_cc < e_lo) | (a_cc + CHUNK > e_hi)

        @pl.when(boundary)
        def _():
            @pl.loop(0, NSTREAM)
            def _(j):
                @pl.loop(0, SW // 16)
                def _(q):
                    glob = a_eff + j * SW + q * 16 + iota16
                    m = (glob >= lo) & (glob < e_hi)
                    valv[j, pl.ds(q * 16, 16)] = jnp.where(
                        m, valv[j, pl.ds(q * 16, 16)], 0.0)
                    rowv[j, pl.ds(q * 16, 16)] = jnp.where(
                        m, rowv[j, pl.ds(q * 16, 16)] - base, DUMMY)

        @pl.when(jnp.logical_not(boundary))
        def _():
            @pl.loop(0, NSTREAM)
            def _(j):
                @pl.loop(0, SW // 16)
                def _(q):
                    rowv[j, pl.ds(q * 16, 16)] = (
                        rowv[j, pl.ds(q * 16, 16)] - base)

        for j in range(NSTREAM):
            @pl.loop(0, SW // 16)
            def _(q, _j=j):
                vv = valv[_j, pl.ds(q * 16, 16)]
                rv = rowv[_j, pl.ds(q * 16, 16)]
                g0 = (bank * NSTREAM + _j) * SW + q * 16
                for e in range(16):
                    sv = vv[e]
                    re = rv[e]
                    plsc.addupdate(acc.at[re, pl.ds(0, 16)],
                                   gath[g0 + e, pl.ds(0, 16)] * sv)
                    plsc.addupdate(acc.at[re, pl.ds(16, 16)],
                                   gath[g0 + e, pl.ds(16, 16)] * sv)

    @pl.loop(0, SW // 16)
    def _(q):
        z = jnp.zeros((16,), _i32)
        colv1[0, pl.ds(q * 16, 16)] = z
        colv1[1, pl.ds(q * 16, 16)] = z
        colv1[2, pl.ds(q * 16, 16)] = z

    issue_gathers(sets[1], 1)
    issue_idx(a0, sets[0])

    def pair_body(i, carry):
        a = a0 + i * (2 * CHUNK)
        wait_idx(sets[0])
        issue_gathers(sets[0], 0)
        wait_gathers(sets[1], 1)
        accum(a - CHUNK, sets[1], 1)
        issue_idx(a + CHUNK, sets[1])
        wait_idx(sets[1])
        issue_gathers(sets[1], 1)
        wait_gathers(sets[0], 0)
        accum(a, sets[0], 0)
        issue_idx(a + 2 * CHUNK, sets[0])
        return carry

    lax.fori_loop(0, n_pairs, pair_body, 0)
    wait_gathers(sets[1], 1)
    accum(a0 + (2 * n_pairs - 1) * CHUNK, sets[1], 1)
    wait_idx(sets[0])

    pltpu.sync_copy(acc.at[pl.ds(0, RPT)], out.at[pl.ds(base, RPT)])


@functools.partial(
    pl.kernel,
    out_type=(jax.ShapeDtypeStruct((BATCH, D), _f32),) * 3,
    mesh=_mesh,
    scratch_types=[
        pltpu.VMEM((SW,), _i32),
        pltpu.VMEM((SW, D), _f32),
        pltpu.VMEM((SW, D), _f32),
        pltpu.VMEM((SW, D), _f32),
        pltpu.VMEM((SW, D), _f32),
        pltpu.VMEM((SW, D), _f32),
        pltpu.SemaphoreType.DMA,
    ],
    compiler_params=_cparams,
)
def _combine(e0, e1, e2, e3, u2d, p2d, n2d, u_out, p_out, n_out,
             idxv, g0, g1, g2, g3, obuf, sem):
    c = lax.axis_index("core")
    s = lax.axis_index("subcore")
    wid = s * NC + c
    for idx2d, dst in ((u2d, u_out), (p2d, p_out), (n2d, n_out)):
        pltpu.sync_copy(idx2d.at[wid], idxv)
        cps = [pltpu.async_copy(t.at[idxv], g, sem)
               for t, g in ((e0, g0), (e1, g1), (e2, g2), (e3, g3))]
        for cp in cps:
            cp.wait()

        @pl.loop(0, SW)
        def _(t):
            for h in (0, 16):
                acc = (g0[t, pl.ds(h, 16)] + g1[t, pl.ds(h, 16)]
                       + g2[t, pl.ds(h, 16)] + g3[t, pl.ds(h, 16)])
                obuf[t, pl.ds(h, 16)] = acc * 0.25

        pltpu.sync_copy(obuf, dst.at[pl.ds(pl.multiple_of(wid * SW, 8), SW)])


def kernel(user_emb, item_emb, adj_val, adj_row, adj_col, users, pos_items, neg_items):
    ego0 = jnp.concatenate([user_emb, item_emb], axis=0)
    splits = jnp.searchsorted(adj_row, jnp.arange(0, NT + 1, RPT)).astype(_i32)
    bounds = jnp.concatenate([splits, jnp.full((48 - NW - 1,), NNZ, _i32)])
    col2d = adj_col.reshape(EROWS, SW)
    row2d = adj_row.reshape(EROWS, SW)
    val2d = adj_val.reshape(EROWS, SW)

    tables = [ego0]
    for _ in range(N_LAYERS):
        tables.append(_spmm_layer(tables[-1], col2d, row2d, val2d, bounds))

    u2d = users.reshape(NW, SW)
    p2d = (pos_items + N_USERS).reshape(NW, SW)
    n2d = (neg_items + N_USERS).reshape(NW, SW)
    return _combine(tables[0], tables[1], tables[2], tables[3], u2d, p2d, n2d)


# speedup vs baseline: 2.9939x; 2.9939x over previous
import functools

import jax
import jax.numpy as jnp
from jax import lax
from jax.experimental import pallas as pl
from jax.experimental.pallas import tpu as pltpu
from jax.experimental.pallas import tpu_sc as plsc

N_USERS = 60000
N_ITEMS = 40000
NT = N_USERS + N_ITEMS
D = 32
NNZ = 1600000
BATCH = 4096
N_LAYERS = 3

NC = 2
NS = 16
RPC = NT // NC
STRIPE = 3128
ACC_ROWS = NS * STRIPE
LAST_ROWS = RPC - (NS - 1) * STRIPE
DUMMY = RPC + 8

CHUNK = 768
SW = 128
NSTREAM = CHUNK // SW
EPAD = NNZ + 4 * CHUNK
EROWS = EPAD // SW

_mesh = plsc.VectorSubcoreMesh(core_axis_name="core", subcore_axis_name="subcore")
_cparams = pltpu.CompilerParams(needs_layout_passes=False,
                                use_tc_tiling_on_sc=False)

_f32 = jnp.float32
_i32 = jnp.int32


@functools.partial(
    pl.kernel,
    out_type=jax.ShapeDtypeStruct((NT, D), _f32),
    mesh=_mesh,
    scratch_types=[
        pltpu.VMEM((16,), _i32),
        pltpu.VMEM((NSTREAM, SW), _i32),
        pltpu.VMEM((NSTREAM, SW), _i32),
        pltpu.VMEM((NSTREAM, SW), _i32),
        pltpu.VMEM((NSTREAM, SW), _i32),
        pltpu.VMEM((NSTREAM, SW), _f32),
        pltpu.VMEM((NSTREAM, SW), _f32),
        pltpu.VMEM((NSTREAM * SW, D), _f32),
        pltpu.VMEM((SW,), _i32),
        pltpu.VMEM_SHARED((ACC_ROWS, D), _f32),
        pltpu.SemaphoreType.DMA,
        pltpu.SemaphoreType.DMA,
        pltpu.SemaphoreType.DMA,
        pltpu.SemaphoreType.DMA,
    ],
    compiler_params=_cparams,
)
def _spmm_layer(table, col2d, row2d, val2d, bounds, out,
                bsm, colv0, colv1, rowv0, rowv1, valv0, valv1, gath, dumv,
                acc, sem_g, sem_s, sem_i0, sem_i1):
    c = lax.axis_index("core")
    s = lax.axis_index("subcore")
    pltpu.sync_copy(bounds, bsm)
    zeros16 = jnp.zeros((16,), _f32)
    iota16 = lax.iota(_i32, 16)
    bvec = bsm[...]
    e_lo = jnp.sum(jnp.where(iota16 == 2 * c, bvec, 0))
    e_hi = jnp.sum(jnp.where(iota16 == 2 * c + 1, bvec, 0))

    GROWS = NSTREAM * SW

    @pl.loop(0, GROWS)
    def _(i):
        gath[i, pl.ds(0, 16)] = zeros16
        gath[i, pl.ds(16, 16)] = zeros16

    for k in range(STRIPE // GROWS):
        pltpu.sync_copy(
            gath, acc.at[pl.ds(pl.multiple_of(s * STRIPE + k * GROWS, 8), GROWS)])
    pltpu.sync_copy(
        gath.at[pl.ds(0, STRIPE % GROWS)],
        acc.at[pl.ds(pl.multiple_of(s * STRIPE + (STRIPE // GROWS) * GROWS, 8),
                     STRIPE % GROWS)])

    base = c * RPC
    per_tile = (e_hi - e_lo + NS - 1) // NS
    my_start = e_lo + s * per_tile
    my_end = jnp.minimum(my_start + per_tile, e_hi)
    a0 = (my_start // CHUNK) * CHUNK
    n_chunks = jnp.maximum(0, (my_end - a0 + CHUNK - 1) // CHUNK)

    bufs = ((colv0, rowv0, valv0, sem_i0), (colv1, rowv1, valv1, sem_i1))

    def issue_idx(a, bset):
        colv, rowv, valv, sem_i = bset
        r = pl.multiple_of(a // SW, NSTREAM)
        pltpu.async_copy(col2d.at[pl.ds(r, NSTREAM)], colv, sem_i)
        pltpu.async_copy(row2d.at[pl.ds(r, NSTREAM)], rowv, sem_i)
        pltpu.async_copy(val2d.at[pl.ds(r, NSTREAM)], valv, sem_i)

    def wait_idx(bset):
        colv, rowv, valv, sem_i = bset
        pltpu.make_async_copy(col2d.at[pl.ds(0, NSTREAM)], colv, sem_i).wait()
        pltpu.make_async_copy(row2d.at[pl.ds(0, NSTREAM)], rowv, sem_i).wait()
        pltpu.make_async_copy(val2d.at[pl.ds(0, NSTREAM)], valv, sem_i).wait()

    def drain_scatters(rowv):
        for j in range(NSTREAM):
            pltpu.make_async_copy(gath.at[pl.ds(j * SW, SW)],
                                  acc.at[rowv.at[j]], sem_s).wait()

    @pl.loop(0, SW // 16)
    def _(q):
        dumv[pl.ds(q * 16, 16)] = jnp.full((16,), DUMMY, _i32)

    for j in range(NSTREAM):
        pltpu.async_copy(gath.at[pl.ds(j * SW, SW)], acc.at[dumv],
                         sem_s, add=True)
    issue_idx(a0, bufs[0])
    plsc.subcore_barrier()

    def do_chunk(a, bset, nbset, pf_a):
        colv, rowv, valv, _ = bset
        wait_idx(bset)
        for j in range(NSTREAM):
            pltpu.make_async_copy(gath.at[pl.ds(j * SW, SW)],
                                  acc.at[rowv.at[j]], sem_s).wait()
            pltpu.async_copy(table.at[colv.at[j]],
                             gath.at[pl.ds(j * SW, SW)], sem_g)
        issue_idx(pf_a, nbset)

        boundary = (a < my_start) | (a + CHUNK > my_end)

        @pl.when(boundary)
        def _():
            @pl.loop(0, NSTREAM)
            def _(j):
                @pl.loop(0, SW // 16)
                def _(q):
                    glob = a + j * SW + q * 16 + iota16
                    m = (glob >= my_start) & (glob < my_end)
                    valv[j, pl.ds(q * 16, 16)] = jnp.where(
                        m, valv[j, pl.ds(q * 16, 16)], 0.0)
                    rowv[j, pl.ds(q * 16, 16)] = jnp.where(
                        m, rowv[j, pl.ds(q * 16, 16)] - base, DUMMY)

        @pl.when(jnp.logical_not(boundary))
        def _():
            @pl.loop(0, NSTREAM)
            def _(j):
                @pl.loop(0, SW // 16)
                def _(q):
                    rowv[j, pl.ds(q * 16, 16)] = rowv[j, pl.ds(q * 16, 16)] - base

        for j in range(NSTREAM):
            pltpu.make_async_copy(table.at[colv.at[j]],
                                  gath.at[pl.ds(j * SW, SW)], sem_g).wait()

            @pl.loop(0, SW // 16)
            def _(q, _j=j):
                vv = valv[_j, pl.ds(q * 16, 16)]
                e0 = _j * SW + q * 16
                for e in range(16):
                    sv = vv[e]
                    gath[e0 + e, pl.ds(0, 16)] = gath[e0 + e, pl.ds(0, 16)] * sv
                    gath[e0 + e, pl.ds(16, 16)] = gath[e0 + e, pl.ds(16, 16)] * sv

            pltpu.async_copy(gath.at[pl.ds(j * SW, SW)],
                             acc.at[rowv.at[j]], sem_s, add=True)

    def pair_body(i, carry):
        a = a0 + i * (2 * CHUNK)
        do_chunk(a, bufs[0], bufs[1], a + CHUNK)
        do_chunk(a + CHUNK, bufs[1], bufs[0], a + 2 * CHUNK)
        return carry

    n_pairs = (n_chunks + 1) // 2
    lax.fori_loop(0, n_pairs, pair_body, 0)
    drain_scatters(rowv1)
    wait_idx(bufs[0])
    plsc.subcore_barrier()

    @pl.when(s < NS - 1)
    def _():
        pltpu.sync_copy(
            acc.at[pl.ds(pl.multiple_of(s * STRIPE, 8), STRIPE)],
            out.at[pl.ds(pl.multiple_of(base + s * STRIPE, 8), STRIPE)])

    @pl.when(s == NS - 1)
    def _():
        pltpu.sync_copy(
            acc.at[pl.ds((NS - 1) * STRIPE, LAST_ROWS)],
            out.at[pl.ds(pl.multiple_of(base + (NS - 1) * STRIPE, 8), LAST_ROWS)])


@functools.partial(
    pl.kernel,
    out_type=(jax.ShapeDtypeStruct((BATCH, D), _f32),) * 3,
    mesh=_mesh,
    scratch_types=[
        pltpu.VMEM((SW,), _i32),
        pltpu.VMEM((SW, D), _f32),
        pltpu.VMEM((SW, D), _f32),
        pltpu.VMEM((SW, D), _f32),
        pltpu.VMEM((SW, D), _f32),
        pltpu.VMEM((SW, D), _f32),
        pltpu.SemaphoreType.DMA,
    ],
    compiler_params=_cparams,
)
def _combine(e0, e1, e2, e3, u2d, p2d, n2d, u_out, p_out, n_out,
             idxv, g0, g1, g2, g3, obuf, sem):
    c = lax.axis_index("core")
    s = lax.axis_index("subcore")
    wid = s * NC + c
    for idx2d, dst in ((u2d, u_out), (p2d, p_out), (n2d, n_out)):
        pltpu.sync_copy(idx2d.at[wid], idxv)
        cps = [pltpu.async_copy(t.at[idxv], g, sem)
               for t, g in ((e0, g0), (e1, g1), (e2, g2), (e3, g3))]
        for cp in cps:
            cp.wait()

        @pl.loop(0, SW)
        def _(t):
            for h in (0, 16):
                acc = (g0[t, pl.ds(h, 16)] + g1[t, pl.ds(h, 16)]
                       + g2[t, pl.ds(h, 16)] + g3[t, pl.ds(h, 16)])
                obuf[t, pl.ds(h, 16)] = acc * 0.25

        pltpu.sync_copy(obuf, dst.at[pl.ds(pl.multiple_of(wid * SW, 8), SW)])


def kernel(user_emb, item_emb, adj_val, adj_row, adj_col, users, pos_items, neg_items):
    ego0 = jnp.concatenate([user_emb, item_emb], axis=0)
    split = jnp.searchsorted(adj_row, RPC).astype(_i32)
    bounds = jnp.zeros((16,), _i32).at[1].set(split).at[2].set(split).at[3].set(jnp.int32(NNZ))
    pad = EPAD - NNZ
    col2d = jnp.pad(adj_col, (0, pad)).reshape(EROWS, SW)
    row2d = jnp.pad(adj_row, (0, pad)).reshape(EROWS, SW)
    val2d = jnp.pad(adj_val, (0, pad)).reshape(EROWS, SW)

    tables = [ego0]
    for _ in range(N_LAYERS):
        tables.append(_spmm_layer(tables[-1], col2d, row2d, val2d, bounds))

    u2d = users.reshape(NC * NS, SW)
    p2d = (pos_items + N_USERS).reshape(NC * NS, SW)
    n2d = (neg_items + N_USERS).reshape(NC * NS, SW)
    return _combine(tables[0], tables[1], tables[2], tables[3], u2d, p2d, n2d)


# speedup vs baseline: 3.0841x; 1.0301x over previous
import functools

import jax
import jax.numpy as jnp
from jax import lax
from jax.experimental import pallas as pl
from jax.experimental.pallas import tpu as pltpu
from jax.experimental.pallas import tpu_sc as plsc

N_USERS = 60000
N_ITEMS = 40000
NT = N_USERS + N_ITEMS
D = 32
NNZ = 1600000
BATCH = 4096
N_LAYERS = 3

NC = 2
NS = 16
RPC = NT // NC
STRIPE = 3128
ACC_ROWS = NS * STRIPE
LAST_ROWS = RPC - (NS - 1) * STRIPE
DUMMY = RPC + 8

CHUNK = 768
SW = 128
NSTREAM = CHUNK // SW
EROWS = NNZ // SW

_mesh = plsc.VectorSubcoreMesh(core_axis_name="core", subcore_axis_name="subcore")
_cparams = pltpu.CompilerParams(needs_layout_passes=False,
                                use_tc_tiling_on_sc=False)

_f32 = jnp.float32
_i32 = jnp.int32


@functools.partial(
    pl.kernel,
    out_type=jax.ShapeDtypeStruct((NT, D), _f32),
    mesh=_mesh,
    scratch_types=[
        pltpu.VMEM((16,), _i32),
        pltpu.VMEM((NSTREAM, SW), _i32),
        pltpu.VMEM((NSTREAM, SW), _i32),
        pltpu.VMEM((NSTREAM, SW), _i32),
        pltpu.VMEM((NSTREAM, SW), _i32),
        pltpu.VMEM((NSTREAM, SW), _f32),
        pltpu.VMEM((NSTREAM, SW), _f32),
        pltpu.VMEM((NSTREAM * SW, D), _f32),
        pltpu.VMEM((SW,), _i32),
        pltpu.VMEM_SHARED((ACC_ROWS, D), _f32),
        pltpu.SemaphoreType.DMA,
        pltpu.SemaphoreType.DMA,
        pltpu.SemaphoreType.DMA,
        pltpu.SemaphoreType.DMA,
    ],
    compiler_params=_cparams,
)
def _spmm_layer(table, col2d, row2d, val2d, bounds, out,
                bsm, colv0, colv1, rowv0, rowv1, valv0, valv1, gath, dumv,
                acc, sem_g, sem_s, sem_i0, sem_i1):
    c = lax.axis_index("core")
    s = lax.axis_index("subcore")
    pltpu.sync_copy(bounds, bsm)
    zeros16 = jnp.zeros((16,), _f32)
    iota16 = lax.iota(_i32, 16)
    bvec = bsm[...]
    e_lo = jnp.sum(jnp.where(iota16 == 2 * c, bvec, 0))
    e_hi = jnp.sum(jnp.where(iota16 == 2 * c + 1, bvec, 0))

    GROWS = NSTREAM * SW

    @pl.loop(0, GROWS)
    def _(i):
        gath[i, pl.ds(0, 16)] = zeros16
        gath[i, pl.ds(16, 16)] = zeros16

    for k in range(STRIPE // GROWS):
        pltpu.sync_copy(
            gath, acc.at[pl.ds(pl.multiple_of(s * STRIPE + k * GROWS, 8), GROWS)])
    pltpu.sync_copy(
        gath.at[pl.ds(0, STRIPE % GROWS)],
        acc.at[pl.ds(pl.multiple_of(s * STRIPE + (STRIPE // GROWS) * GROWS, 8),
                     STRIPE % GROWS)])

    base = c * RPC
    per_tile = (e_hi - e_lo + NS - 1) // NS
    my_start = e_lo + s * per_tile
    my_end = jnp.minimum(my_start + per_tile, e_hi)
    a0 = (my_start // CHUNK) * CHUNK
    n_chunks = jnp.maximum(0, (my_end - a0 + CHUNK - 1) // CHUNK)

    bufs = ((colv0, rowv0, valv0, sem_i0), (colv1, rowv1, valv1, sem_i1))

    def issue_idx(a, bset):
        colv, rowv, valv, sem_i = bset
        r = jnp.minimum(a, NNZ - CHUNK) // SW
        pltpu.async_copy(col2d.at[pl.ds(r, NSTREAM)], colv, sem_i)
        pltpu.async_copy(row2d.at[pl.ds(r, NSTREAM)], rowv, sem_i)
        pltpu.async_copy(val2d.at[pl.ds(r, NSTREAM)], valv, sem_i)

    def wait_idx(bset):
        colv, rowv, valv, sem_i = bset
        pltpu.make_async_copy(col2d.at[pl.ds(0, NSTREAM)], colv, sem_i).wait()
        pltpu.make_async_copy(row2d.at[pl.ds(0, NSTREAM)], rowv, sem_i).wait()
        pltpu.make_async_copy(val2d.at[pl.ds(0, NSTREAM)], valv, sem_i).wait()

    def drain_scatters(rowv):
        for j in range(NSTREAM):
            pltpu.make_async_copy(gath.at[pl.ds(j * SW, SW)],
                                  acc.at[rowv.at[j]], sem_s).wait()

    @pl.loop(0, SW // 16)
    def _(q):
        dumv[pl.ds(q * 16, 16)] = jnp.full((16,), DUMMY, _i32)

    for j in range(NSTREAM):
        pltpu.async_copy(gath.at[pl.ds(j * SW, SW)], acc.at[dumv],
                         sem_s, add=True)
    issue_idx(a0, bufs[0])
    plsc.subcore_barrier()

    def do_chunk(a, bset, nbset, pf_a):
        colv, rowv, valv, _ = bset
        wait_idx(bset)
        for j in range(NSTREAM):
            pltpu.make_async_copy(gath.at[pl.ds(j * SW, SW)],
                                  acc.at[rowv.at[j]], sem_s).wait()
            pltpu.async_copy(table.at[colv.at[j]],
                             gath.at[pl.ds(j * SW, SW)], sem_g)
        issue_idx(pf_a, nbset)

        boundary = (a < my_start) | (a + CHUNK > my_end)
        a_eff = jnp.minimum(a, NNZ - CHUNK)
        lo = jnp.maximum(my_start, a)

        @pl.when(boundary)
        def _():
            @pl.loop(0, NSTREAM)
            def _(j):
                @pl.loop(0, SW // 16)
                def _(q):
                    glob = a_eff + j * SW + q * 16 + iota16
                    m = (glob >= lo) & (glob < my_end)
                    valv[j, pl.ds(q * 16, 16)] = jnp.where(
                        m, valv[j, pl.ds(q * 16, 16)], 0.0)
                    rowv[j, pl.ds(q * 16, 16)] = jnp.where(
                        m, rowv[j, pl.ds(q * 16, 16)] - base, DUMMY)

        @pl.when(jnp.logical_not(boundary))
        def _():
            @pl.loop(0, NSTREAM)
            def _(j):
                @pl.loop(0, SW // 16)
                def _(q):
                    rowv[j, pl.ds(q * 16, 16)] = rowv[j, pl.ds(q * 16, 16)] - base

        for j in range(NSTREAM):
            pltpu.make_async_copy(table.at[colv.at[j]],
                                  gath.at[pl.ds(j * SW, SW)], sem_g).wait()

            @pl.loop(0, SW // 16)
            def _(q, _j=j):
                vv = valv[_j, pl.ds(q * 16, 16)]
                e0 = _j * SW + q * 16
                for e in range(16):
                    sv = vv[e]
                    gath[e0 + e, pl.ds(0, 16)] = gath[e0 + e, pl.ds(0, 16)] * sv
                    gath[e0 + e, pl.ds(16, 16)] = gath[e0 + e, pl.ds(16, 16)] * sv

            pltpu.async_copy(gath.at[pl.ds(j * SW, SW)],
                             acc.at[rowv.at[j]], sem_s, add=True)

    def pair_body(i, carry):
        a = a0 + i * (2 * CHUNK)
        do_chunk(a, bufs[0], bufs[1], a + CHUNK)
        do_chunk(a + CHUNK, bufs[1], bufs[0], a + 2 * CHUNK)
        return carry

    n_pairs = (n_chunks + 1) // 2
    lax.fori_loop(0, n_pairs, pair_body, 0)
    drain_scatters(rowv1)
    wait_idx(bufs[0])
    plsc.subcore_barrier()

    @pl.when(s < NS - 1)
    def _():
        pltpu.sync_copy(
            acc.at[pl.ds(pl.multiple_of(s * STRIPE, 8), STRIPE)],
            out.at[pl.ds(pl.multiple_of(base + s * STRIPE, 8), STRIPE)])

    @pl.when(s == NS - 1)
    def _():
        pltpu.sync_copy(
            acc.at[pl.ds((NS - 1) * STRIPE, LAST_ROWS)],
            out.at[pl.ds(pl.multiple_of(base + (NS - 1) * STRIPE, 8), LAST_ROWS)])


@functools.partial(
    pl.kernel,
    out_type=(jax.ShapeDtypeStruct((BATCH, D), _f32),) * 3,
    mesh=_mesh,
    scratch_types=[
        pltpu.VMEM((SW,), _i32),
        pltpu.VMEM((SW, D), _f32),
        pltpu.VMEM((SW, D), _f32),
        pltpu.VMEM((SW, D), _f32),
        pltpu.VMEM((SW, D), _f32),
        pltpu.VMEM((SW, D), _f32),
        pltpu.SemaphoreType.DMA,
    ],
    compiler_params=_cparams,
)
def _combine(e0, e1, e2, e3, u2d, p2d, n2d, u_out, p_out, n_out,
             idxv, g0, g1, g2, g3, obuf, sem):
    c = lax.axis_index("core")
    s = lax.axis_index("subcore")
    wid = s * NC + c
    for idx2d, dst in ((u2d, u_out), (p2d, p_out), (n2d, n_out)):
        pltpu.sync_copy(idx2d.at[wid], idxv)
        cps = [pltpu.async_copy(t.at[idxv], g, sem)
               for t, g in ((e0, g0), (e1, g1), (e2, g2), (e3, g3))]
        for cp in cps:
            cp.wait()

        @pl.loop(0, SW)
        def _(t):
            for h in (0, 16):
                acc = (g0[t, pl.ds(h, 16)] + g1[t, pl.ds(h, 16)]
                       + g2[t, pl.ds(h, 16)] + g3[t, pl.ds(h, 16)])
                obuf[t, pl.ds(h, 16)] = acc * 0.25

        pltpu.sync_copy(obuf, dst.at[pl.ds(pl.multiple_of(wid * SW, 8), SW)])


def kernel(user_emb, item_emb, adj_val, adj_row, adj_col, users, pos_items, neg_items):
    ego0 = jnp.concatenate([user_emb, item_emb], axis=0)
    split = jnp.searchsorted(adj_row, RPC).astype(_i32)
    bounds = jnp.zeros((16,), _i32).at[1].set(split).at[2].set(split).at[3].set(jnp.int32(NNZ))
    col2d = adj_col.reshape(EROWS, SW)
    row2d = adj_row.reshape(EROWS, SW)
    val2d = adj_val.reshape(EROWS, SW)

    tables = [ego0]
    for _ in range(N_LAYERS):
        tables.append(_spmm_layer(tables[-1], col2d, row2d, val2d, bounds))

    u2d = users.reshape(NC * NS, SW)
    p2d = (pos_items + N_USERS).reshape(NC * NS, SW)
    n2d = (neg_items + N_USERS).reshape(NC * NS, SW)
    return _combine(tables[0], tables[1], tables[2], tables[3], u2d, p2d, n2d)
